# baseline (speedup 1.0000x reference)
import functools

import jax
import jax.numpy as jnp
from jax import lax
from jax.experimental import pallas as pl
from jax.experimental.pallas import tpu as pltpu
from jax.experimental.pallas import tpu_sc as plsc

_NC = 2
_NS = 16
_LANES = 16
_EB = 128
_F = 128

_MESH = plsc.VectorSubcoreMesh(
    core_axis_name="c", subcore_axis_name="s", num_cores=_NC, num_subcores=_NS
)


def _pad_edges(src, dst, n_nodes):
    e = src.shape[0]
    ept = -(-e // _NS)
    ept = -(-ept // _EB) * _EB
    ep = ept * _NS
    pad = ep - e
    srcp = jnp.concatenate([src, jnp.zeros((pad,), jnp.int32)])
    dstp = jnp.concatenate([dst, jnp.full((pad,), n_nodes, jnp.int32)])
    return srcp, dstp, ep


def _acc_rows(n_nodes):
    zr = 16 * (-(-(n_nodes + 1) // (16 * _NS)))
    return zr, zr * _NS


def _make_segsum(n_chunks, n_nodes, ep):
    assert n_chunks % _NC == 0 and n_nodes % _NS == 0
    rounds = n_chunks // _NC
    ept = ep // _NS
    nblk = ept // _EB
    zr, n_acc = _acc_rows(n_nodes)

    @functools.partial(
        pl.kernel,
        out_type=jax.ShapeDtypeStruct((n_chunks * n_acc, _F), jnp.float32),
        mesh=_MESH,
        scratch_types=[
            pltpu.VMEM((_EB,), jnp.int32),
            pltpu.VMEM((_EB,), jnp.int32),
            pltpu.VMEM((_EB,), jnp.int32),
            pltpu.VMEM((_EB, _F), jnp.float32),
            pltpu.VMEM((16, _F), jnp.float32),
            pltpu.VMEM_SHARED((n_acc, _F), jnp.float32),
            pltpu.SemaphoreType.DMA,
        ],
    )
    def segsum(y_ref, src_ref, dst_ref, out_ref, idx_s, idx_s2, idx_d,
               rows, zbuf, acc, gsem):
        c = lax.axis_index("c")
        s = lax.axis_index("s")
        for i in range(16):
            for j in range(_F // _LANES):
                zbuf[i, pl.ds(j * _LANES, _LANES)] = jnp.zeros((_LANES,), jnp.float32)

        for r in range(rounds):
            chunk = r * _NC + c
            base = chunk * n_nodes

            def _zero(z, _):
                pltpu.sync_copy(zbuf, acc.at[pl.ds(s * zr + z * 16, 16)])
                return ()
            lax.fori_loop(0, zr // 16, _zero, ())

            plsc.subcore_barrier()

            def _blk(b, _):
                off = s * ept + b * _EB
                pltpu.sync_copy(src_ref.at[pl.ds(off, _EB)], idx_s)
                pltpu.sync_copy(dst_ref.at[pl.ds(off, _EB)], idx_d)
                for j in range(_EB // _LANES):
                    sl = pl.ds(j * _LANES, _LANES)
                    idx_s2[sl] = idx_s[sl] + base
                pltpu.async_copy(y_ref.at[idx_s2], rows, gsem).wait()
                pltpu.sync_copy(rows, acc.at[idx_d], add=True)
                return ()
            lax.fori_loop(0, nblk, _blk, ())
            plsc.subcore_barrier()

            pltpu.sync_copy(
                acc.at[pl.ds(s * zr, zr)],
                out_ref.at[pl.ds(chunk * n_acc + s * zr, zr)],
            )
            plsc.subcore_barrier()

    return segsum


def _make_count(n_nodes, ep):
    ept = ep // _NS
    nblk = ept // _EB
    zr, n_acc = _acc_rows(n_nodes)

    @functools.partial(
        pl.kernel,
        out_type=jax.ShapeDtypeStruct((n_acc, _F), jnp.float32),
        mesh=_MESH,
        scratch_types=[
            pltpu.VMEM((_EB,), jnp.int32),
            pltpu.VMEM((_EB, _F), jnp.float32),
            pltpu.VMEM((16, _F), jnp.float32),
            pltpu.VMEM_SHARED((n_acc, _F), jnp.float32),
        ],
    )
    def count(dst_ref, out_ref, idx_d, ones, zbuf, acc):
        c = lax.axis_index("c")
        s = lax.axis_index("s")

        @pl.when(c == 0)
        def _():
            for i in range(_EB):
                for j in range(_F // _LANES):
                    ones[i, pl.ds(j * _LANES, _LANES)] = jnp.ones(
                        (_LANES,), jnp.float32)
            for i in range(16):
                for j in range(_F // _LANES):
                    zbuf[i, pl.ds(j * _LANES, _LANES)] = jnp.zeros(
                        (_LANES,), jnp.float32)
            def _zero(z, _):
                pltpu.sync_copy(zbuf, acc.at[pl.ds(s * zr + z * 16, 16)])
                return ()
            lax.fori_loop(0, zr // 16, _zero, ())
            plsc.subcore_barrier()

            def _edges(b, _):
                off = s * ept + b * _EB
                pltpu.sync_copy(dst_ref.at[pl.ds(off, _EB)], idx_d)
                pltpu.sync_copy(ones, acc.at[idx_d], add=True)
                return ()
            lax.fori_loop(0, nblk, _edges, ())
            plsc.subcore_barrier()

            pltpu.sync_copy(acc.at[pl.ds(s * zr, zr)],
                            out_ref.at[pl.ds(s * zr, zr)])

    return count


_NB = 1000
_G = 64


def _deg_inv_kernel(batch_ref, o_ref):
    i = pl.program_id(0)
    nb = pl.num_programs(0)

    @pl.when(i == 0)
    def _():
        o_ref[...] = jnp.zeros_like(o_ref)

    oh = (batch_ref[...] == lax.broadcasted_iota(jnp.int32, (1, _G), 1)
          ).astype(jnp.float32)
    o_ref[...] += jnp.sum(oh, axis=0, keepdims=True)

    @pl.when(i == nb - 1)
    def _():
        deg = o_ref[...]
        o_ref[...] = jnp.where(
            deg > 0.0, lax.rsqrt(jnp.maximum(deg, 1.0)), 0.0)


def _deg_inv(batch2, n_nodes):
    return pl.pallas_call(
        _deg_inv_kernel,
        grid=(n_nodes // _NB,),
        in_specs=[pl.BlockSpec((_NB, 1), lambda i: (i, 0))],
        out_specs=pl.BlockSpec((1, _G), lambda i: (0, 0)),
        out_shape=jax.ShapeDtypeStruct((1, _G), jnp.float32),
    )(batch2)


def _xwr_kernel(x_ref, wr_ref, o_ref):
    cx = x_ref.shape[0]
    x = jnp.concatenate([x_ref[k] for k in range(cx)], axis=1)
    o_ref[...] = jnp.dot(x, wr_ref[...], preferred_element_type=jnp.float32)


def _xwr(xch, wr, n_nodes, h):
    cx = xch.shape[0]
    return pl.pallas_call(
        _xwr_kernel,
        grid=(n_nodes // _NB,),
        in_specs=[
            pl.BlockSpec((cx, _NB, _F), lambda i: (0, i, 0)),
            pl.BlockSpec(wr.shape, lambda i: (0, 0)),
        ],
        out_specs=pl.BlockSpec((_NB, h), lambda i: (i, 0)),
        out_shape=jax.ShapeDtypeStruct((n_nodes, h), jnp.float32),
    )(xch, wr)


def _layer_kernel(agg_ref, xwr_ref, cnt_ref, batch_ref, wl_ref, bl_ref,
                  inv_ref, t_ref, st_ref):
    i = pl.program_id(0)
    ca = agg_ref.shape[0]
    agg = jnp.concatenate([agg_ref[k] for k in range(ca)], axis=1)
    rinv = 1.0 / jnp.maximum(cnt_ref[...], 1.0)
    mean = agg * rinv
    u = (jnp.dot(mean, wl_ref[...], preferred_element_type=jnp.float32)
         + xwr_ref[...] + bl_ref[...])
    r = jnp.maximum(u, 0.0)
    oh = (batch_ref[...] == lax.broadcasted_iota(jnp.int32, (_NB, _G), 1)
          ).astype(jnp.float32)
    scale = jnp.sum(oh * inv_ref[...], axis=1, keepdims=True)
    t = r * scale
    t_ref[...] = t

    @pl.when(i == 0)
    def _():
        st_ref[...] = jnp.zeros_like(st_ref)

    st_ref[0:1, :] += jnp.sum(t, axis=0, keepdims=True)
    st_ref[1:2, :] += jnp.sum(t * t, axis=0, keepdims=True)


def _layer(agg, xwr, cnt, batch2, wl, bl, inv, n_nodes, h):
    ca = agg.shape[0]
    return pl.pallas_call(
        _layer_kernel,
        grid=(n_nodes // _NB,),
        in_specs=[
            pl.BlockSpec((ca, _NB, _F), lambda i: (0, i, 0)),
            pl.BlockSpec((_NB, h), lambda i: (i, 0)),
            pl.BlockSpec((_NB, 1), lambda i: (i, 0)),
            pl.BlockSpec((_NB, 1), lambda i: (i, 0)),
            pl.BlockSpec(wl.shape, lambda i: (0, 0)),
            pl.BlockSpec((1, h), lambda i: (0, 0)),
            pl.BlockSpec((1, _G), lambda i: (0, 0)),
        ],
        out_specs=[
            pl.BlockSpec((_NB, h), lambda i: (i, 0)),
            pl.BlockSpec((2, h), lambda i: (0, 0)),
        ],
        out_shape=[
            jax.ShapeDtypeStruct((n_nodes, h), jnp.float32),
            jax.ShapeDtypeStruct((2, h), jnp.float32),
        ],
    )(agg, xwr, cnt, batch2, wl, bl, inv)


def _norm_kernel(n_nodes, t_ref, st_ref, g_ref, b_ref, o_ref):
    mu = st_ref[0:1, :] / n_nodes
    var = st_ref[1:2, :] / n_nodes - mu * mu
    rstd = lax.rsqrt(var + 1e-5)
    y = (t_ref[...] - mu) * (rstd * g_ref[...]) + b_ref[...]
    nch = o_ref.shape[0]
    for k in range(nch):
        o_ref[k] = y[:, k * _F:(k + 1) * _F]


def _norm(t, st, gamma, beta, n_nodes, h):
    nch = h // _F
    return pl.pallas_call(
        functools.partial(_norm_kernel, float(n_nodes)),
        grid=(n_nodes // _NB,),
        in_specs=[
            pl.BlockSpec((_NB, h), lambda i: (i, 0)),
            pl.BlockSpec((2, h), lambda i: (0, 0)),
            pl.BlockSpec((1, h), lambda i: (0, 0)),
            pl.BlockSpec((1, h), lambda i: (0, 0)),
        ],
        out_specs=pl.BlockSpec((nch, _NB, _F), lambda i: (0, i, 0)),
        out_shape=jax.ShapeDtypeStruct((nch, n_nodes, _F), jnp.float32),
    )(t, st, gamma, beta)


def _gate_kernel(x_ref, batch_ref, wg_ref, bg_ref, gate_ref, m_ref):
    i = pl.program_id(0)
    nc = x_ref.shape[0]
    x = jnp.concatenate([x_ref[k] for k in range(nc)], axis=1)
    g = jnp.dot(x, wg_ref[...], preferred_element_type=jnp.float32) + bg_ref[...]
    gate_ref[...] = g
    ohb = batch_ref[...] == lax.broadcasted_iota(jnp.int32, (_NB, _G), 1)
    wh = jnp.where(ohb, g, -jnp.inf)
    blkmax = jnp.max(wh, axis=0, keepdims=True)

    @pl.when(i == 0)
    def _():
        m_ref[...] = jnp.full_like(m_ref, -jnp.inf)

    m_ref[...] = jnp.maximum(m_ref[...], blkmax)


def _gate(xch, batch2, wg, bg, n_nodes, h):
    nch = h // _F
    return pl.pallas_call(
        _gate_kernel,
        grid=(n_nodes // _NB,),
        in_specs=[
            pl.BlockSpec((nch, _NB, _F), lambda i: (0, i, 0)),
            pl.BlockSpec((_NB, 1), lambda i: (i, 0)),
            pl.BlockSpec((h, 1), lambda i: (0, 0)),
            pl.BlockSpec((1, 1), lambda i: (0, 0)),
        ],
        out_specs=[
            pl.BlockSpec((_NB, 1), lambda i: (i, 0)),
            pl.BlockSpec((1, _G), lambda i: (0, 0)),
        ],
        out_shape=[
            jax.ShapeDtypeStruct((n_nodes, 1), jnp.float32),
            jax.ShapeDtypeStruct((1, _G), jnp.float32),
        ],
    )(xch, batch2, wg, bg)


def _pool_kernel(x_ref, batch_ref, gate_ref, m_ref, u_ref, s_ref):
    i = pl.program_id(0)
    nc = x_ref.shape[0]
    x = jnp.concatenate([x_ref[k] for k in range(nc)], axis=1)
    m = m_ref[...]
    m = jnp.where(jnp.isfinite(m), m, 0.0)
    oh = (batch_ref[...] == lax.broadcasted_iota(jnp.int32, (_NB, _G), 1)
          ).astype(jnp.float32)
    mg = jnp.sum(oh * m, axis=1, keepdims=True)
    e = jnp.exp(gate_ref[...] - mg)
    we = oh * e

    @pl.when(i == 0)
    def _():
        u_ref[...] = jnp.zeros_like(u_ref)
        s_ref[...] = jnp.zeros_like(s_ref)

    u_ref[...] += lax.dot_general(we, x, (((0,), (0,)), ((), ())),
                                  preferred_element_type=jnp.float32)
    ones = jnp.ones((_NB, 1), jnp.float32)
    s_ref[...] += lax.dot_general(we, ones, (((0,), (0,)), ((), ())),
                                  preferred_element_type=jnp.float32)


def _pool(xch, batch2, gate, m, n_nodes, h):
    nch = h // _F
    return pl.pallas_call(
        _pool_kernel,
        grid=(n_nodes // _NB,),
        in_specs=[
            pl.BlockSpec((nch, _NB, _F), lambda i: (0, i, 0)),
            pl.BlockSpec((_NB, 1), lambda i: (i, 0)),
            pl.BlockSpec((_NB, 1), lambda i: (i, 0)),
            pl.BlockSpec((1, _G), lambda i: (0, 0)),
        ],
        out_specs=[
            pl.BlockSpec((_G, h), lambda i: (0, 0)),
            pl.BlockSpec((_G, 1), lambda i: (0, 0)),
        ],
        out_shape=[
            jax.ShapeDtypeStruct((_G, h), jnp.float32),
            jax.ShapeDtypeStruct((_G, 1), jnp.float32),
        ],
    )(xch, batch2, gate, m)


def _mlp_kernel(u_ref, s_ref, w1_ref, b1_ref, w2_ref, b2_ref, o_ref):
    p = u_ref[...] / jnp.maximum(s_ref[...], 1e-16)
    hdn = jnp.maximum(
        jnp.dot(p, w1_ref[...], preferred_element_type=jnp.float32)
        + b1_ref[...], 0.0)
    o_ref[...] = (jnp.dot(hdn, w2_ref[...], preferred_element_type=jnp.float32)
                  + b2_ref[...])


def _mlp(u, s, w1, b1, w2, b2, h, dout):
    return pl.pallas_call(
        _mlp_kernel,
        out_shape=jax.ShapeDtypeStruct((_G, dout), jnp.float32),
    )(u, s, w1, b1, w2, b2)


def kernel(x, edge_index, batch, Wl0, bl0, Wr0, gamma0, beta0, Wl1, bl1, Wr1,
           gamma1, beta1, Wl2, bl2, Wr2, gamma2, beta2, Wg, bg, W1, b1, W2,
           b2):
    n, din = x.shape
    h = Wl0.shape[1]
    dout = W2.shape[1]
    src, dst = edge_index[0], edge_index[1]
    srcp, dstp, ep = _pad_edges(src, dst, n)
    batch2 = batch.reshape(n, 1)

    _, n_acc = _acc_rows(n)
    inv = _deg_inv(batch2, n)
    cnt = _make_count(n, ep)(dstp)[:n, :1]

    xch = x.reshape(n, din // _F, _F).transpose(1, 0, 2)

    layers = [(Wl0, bl0, Wr0, gamma0, beta0),
              (Wl1, bl1, Wr1, gamma1, beta1),
              (Wl2, bl2, Wr2, gamma2, beta2)]
    for (wl, bl, wr, gamma, beta) in layers:
        c = xch.shape[0]
        aggf = _make_segsum(c, n, ep)(xch.reshape(c * n, _F), srcp, dstp)
        xwr = _xwr(xch, wr, n, h)
        agg = aggf.reshape(c, n_acc, _F)[:, :n, :]
        t, st = _layer(agg, xwr, cnt, batch2, wl, bl.reshape(1, h), inv,
                       n, h)
        xch = _norm(t, st, gamma.reshape(1, h), beta.reshape(1, h), n, h)

    gate, m = _gate(xch, batch2, Wg, bg.reshape(1, 1), n, h)
    u, s = _pool(xch, batch2, gate, m, n, h)
    return _mlp(u, s, W1, b1.reshape(1, h), W2, b2.reshape(1, dout), h, dout)


# speedup vs baseline: 1.0064x; 1.0064x over previous
import functools

import jax
import jax.numpy as jnp
from jax import lax
from jax.experimental import pallas as pl
from jax.experimental.pallas import tpu as pltpu
from jax.experimental.pallas import tpu_sc as plsc

_NC = 2
_NS = 16
_LANES = 16
_EB = 128
_F = 128

_MESH = plsc.VectorSubcoreMesh(
    core_axis_name="c", subcore_axis_name="s", num_cores=_NC, num_subcores=_NS
)


def _pad_edges(src, dst, n_nodes):
    e = src.shape[0]
    ept = -(-e // _NS)
    ept = -(-ept // _EB) * _EB
    ep = ept * _NS
    pad = ep - e
    srcp = jnp.concatenate([src, jnp.zeros((pad,), jnp.int32)])
    dstp = jnp.concatenate([dst, jnp.full((pad,), n_nodes, jnp.int32)])
    return srcp, dstp, ep


def _acc_rows(n_nodes):
    zr = 16 * (-(-(n_nodes + 1) // (16 * _NS)))
    return zr, zr * _NS


def _make_segsum(n_chunks, n_nodes, ep):
    assert n_chunks % _NC == 0 and n_nodes % _NS == 0
    rounds = n_chunks // _NC
    ept = ep // _NS
    nblk = ept // _EB
    zr, n_acc = _acc_rows(n_nodes)

    @functools.partial(
        pl.kernel,
        out_type=jax.ShapeDtypeStruct((n_chunks * n_acc, _F), jnp.float32),
        mesh=_MESH,
        scratch_types=[
            pltpu.VMEM((_EB,), jnp.int32),
            pltpu.VMEM((_EB,), jnp.int32),
            pltpu.VMEM((_EB,), jnp.int32),
            pltpu.VMEM((_EB, _F), jnp.float32),
            pltpu.VMEM((16, _F), jnp.float32),
            pltpu.VMEM_SHARED((n_acc, _F), jnp.float32),
            pltpu.SemaphoreType.DMA,
        ],
    )
    def segsum(y_ref, src_ref, dst_ref, out_ref, idx_s, idx_s2, idx_d,
               rows, zbuf, acc, gsem):
        c = lax.axis_index("c")
        s = lax.axis_index("s")
        for i in range(16):
            for j in range(_F // _LANES):
                zbuf[i, pl.ds(j * _LANES, _LANES)] = jnp.zeros((_LANES,), jnp.float32)

        for r in range(rounds):
            chunk = r * _NC + c
            base = chunk * n_nodes

            def _zero(z, _):
                pltpu.sync_copy(zbuf, acc.at[pl.ds(s * zr + z * 16, 16)])
                return ()
            lax.fori_loop(0, zr // 16, _zero, ())

            plsc.subcore_barrier()

            def _blk(b, _):
                off = s * ept + b * _EB
                pltpu.sync_copy(src_ref.at[pl.ds(off, _EB)], idx_s)
                pltpu.sync_copy(dst_ref.at[pl.ds(off, _EB)], idx_d)
                for j in range(_EB // _LANES):
                    sl = pl.ds(j * _LANES, _LANES)
                    idx_s2[sl] = idx_s[sl] + base
                pltpu.async_copy(y_ref.at[idx_s2], rows, gsem).wait()
                pltpu.sync_copy(rows, acc.at[idx_d], add=True)
                return ()
            lax.fori_loop(0, nblk, _blk, ())
            plsc.subcore_barrier()

            pltpu.sync_copy(
                acc.at[pl.ds(s * zr, zr)],
                out_ref.at[pl.ds(chunk * n_acc + s * zr, zr)],
            )
            plsc.subcore_barrier()

    return segsum


def _make_count(n_nodes, ep):
    ept = ep // _NS
    nblk = ept // _EB
    zr, n_acc = _acc_rows(n_nodes)

    @functools.partial(
        pl.kernel,
        out_type=jax.ShapeDtypeStruct((n_acc, _F), jnp.float32),
        mesh=_MESH,
        scratch_types=[
            pltpu.VMEM((_EB,), jnp.int32),
            pltpu.VMEM((_EB, _F), jnp.float32),
            pltpu.VMEM((16, _F), jnp.float32),
            pltpu.VMEM_SHARED((n_acc, _F), jnp.float32),
        ],
    )
    def count(dst_ref, out_ref, idx_d, ones, zbuf, acc):
        c = lax.axis_index("c")
        s = lax.axis_index("s")

        @pl.when(c == 0)
        def _():
            for i in range(_EB):
                for j in range(_F // _LANES):
                    ones[i, pl.ds(j * _LANES, _LANES)] = jnp.ones(
                        (_LANES,), jnp.float32)
            for i in range(16):
                for j in range(_F // _LANES):
                    zbuf[i, pl.ds(j * _LANES, _LANES)] = jnp.zeros(
                        (_LANES,), jnp.float32)
            def _zero(z, _):
                pltpu.sync_copy(zbuf, acc.at[pl.ds(s * zr + z * 16, 16)])
                return ()
            lax.fori_loop(0, zr // 16, _zero, ())
            plsc.subcore_barrier()

            def _edges(b, _):
                off = s * ept + b * _EB
                pltpu.sync_copy(dst_ref.at[pl.ds(off, _EB)], idx_d)
                pltpu.sync_copy(ones, acc.at[idx_d], add=True)
                return ()
            lax.fori_loop(0, nblk, _edges, ())
            plsc.subcore_barrier()

            pltpu.sync_copy(acc.at[pl.ds(s * zr, zr)],
                            out_ref.at[pl.ds(s * zr, zr)])

    return count


_NB = 1000
_G = 64


def _deg_inv_kernel(batch_ref, o_ref):
    i = pl.program_id(0)
    nb = pl.num_programs(0)

    @pl.when(i == 0)
    def _():
        o_ref[...] = jnp.zeros_like(o_ref)

    oh = (batch_ref[...] == lax.broadcasted_iota(jnp.int32, (1, _G), 1)
          ).astype(jnp.float32)
    o_ref[...] += jnp.sum(oh, axis=0, keepdims=True)

    @pl.when(i == nb - 1)
    def _():
        deg = o_ref[...]
        o_ref[...] = jnp.where(
            deg > 0.0, lax.rsqrt(jnp.maximum(deg, 1.0)), 0.0)


def _deg_inv(batch2, n_nodes):
    return pl.pallas_call(
        _deg_inv_kernel,
        grid=(n_nodes // _NB,),
        in_specs=[pl.BlockSpec((_NB, 1), lambda i: (i, 0))],
        out_specs=pl.BlockSpec((1, _G), lambda i: (0, 0)),
        out_shape=jax.ShapeDtypeStruct((1, _G), jnp.float32),
    )(batch2)


def _bn_affine(st_ref, g_ref, b_ref, n_nodes):
    mu = st_ref[0:1, :] / n_nodes
    var = st_ref[1:2, :] / n_nodes - mu * mu
    a = g_ref[...] * lax.rsqrt(var + 1e-5)
    bb = b_ref[...] - mu * a
    return a, bb


def _xwr_kernel(n_nodes, x_ref, st_ref, g_ref, b_ref, wr_ref, o_ref):
    cx = x_ref.shape[0]
    x = jnp.concatenate([x_ref[k] for k in range(cx)], axis=1)
    if st_ref is not None:
        a, bb = _bn_affine(st_ref, g_ref, b_ref, n_nodes)
        x = x * a + bb
    o_ref[...] = jnp.dot(x, wr_ref[...], preferred_element_type=jnp.float32)


def _xwr(xch, st, gamma, beta, wr, n_nodes, h):
    cx = xch.shape[0]
    stats_specs = [] if st is None else [
        pl.BlockSpec((2, xch.shape[0] * _F), lambda i: (0, 0)),
        pl.BlockSpec((1, xch.shape[0] * _F), lambda i: (0, 0)),
        pl.BlockSpec((1, xch.shape[0] * _F), lambda i: (0, 0)),
    ]
    args = (xch,) if st is None else (xch, st, gamma, beta)
    body = (functools.partial(_xwr_kernel, float(n_nodes))
            if st is not None else
            (lambda x_ref, wr_ref, o_ref:
             _xwr_kernel(float(n_nodes), x_ref, None, None, None, wr_ref,
                         o_ref)))
    return pl.pallas_call(
        body,
        grid=(n_nodes // _NB,),
        in_specs=[
            pl.BlockSpec((cx, _NB, _F), lambda i: (0, i, 0)),
            *stats_specs,
            pl.BlockSpec(wr.shape, lambda i: (0, 0)),
        ],
        out_specs=pl.BlockSpec((_NB, h), lambda i: (i, 0)),
        out_shape=jax.ShapeDtypeStruct((n_nodes, h), jnp.float32),
    )(*args, wr)


def _layer_kernel(n_nodes, agg_ref, st_in, g_in, b_in, xwr_ref, cnt_ref,
                  batch_ref, wl_ref, bl_ref, inv_ref, t_ref, st_ref):
    i = pl.program_id(0)
    ca = agg_ref.shape[0]
    agg = jnp.concatenate([agg_ref[k] for k in range(ca)], axis=1)
    rinv = 1.0 / jnp.maximum(cnt_ref[...], 1.0)
    mean = agg * rinv
    if st_in is not None:
        a, bb = _bn_affine(st_in, g_in, b_in, n_nodes)
        mean = mean * a + bb * jnp.minimum(cnt_ref[...], 1.0)
    u = (jnp.dot(mean, wl_ref[...], preferred_element_type=jnp.float32)
         + xwr_ref[...] + bl_ref[...])
    r = jnp.maximum(u, 0.0)
    oh = (batch_ref[...] == lax.broadcasted_iota(jnp.int32, (_NB, _G), 1)
          ).astype(jnp.float32)
    scale = jnp.sum(oh * inv_ref[...], axis=1, keepdims=True)
    t = r * scale
    nch = t_ref.shape[0]
    for k in range(nch):
        t_ref[k] = t[:, k * _F:(k + 1) * _F]

    @pl.when(i == 0)
    def _():
        st_ref[...] = jnp.zeros_like(st_ref)

    st_ref[0:1, :] += jnp.sum(t, axis=0, keepdims=True)
    st_ref[1:2, :] += jnp.sum(t * t, axis=0, keepdims=True)


def _layer(agg, st, gamma, beta, xwr, cnt, batch2, wl, bl, inv, n_nodes, h):
    ca = agg.shape[0]
    nch = h // _F
    stats_specs = [] if st is None else [
        pl.BlockSpec((2, h), lambda i: (0, 0)),
        pl.BlockSpec((1, h), lambda i: (0, 0)),
        pl.BlockSpec((1, h), lambda i: (0, 0)),
    ]
    args = (agg,) if st is None else (agg, st, gamma, beta)
    if st is not None:
        body = functools.partial(_layer_kernel, float(n_nodes))
    else:
        def body(agg_ref, xwr_ref, cnt_ref, batch_ref, wl_ref, bl_ref,
                 inv_ref, t_ref, st_ref):
            _layer_kernel(float(n_nodes), agg_ref, None, None, None, xwr_ref,
                          cnt_ref, batch_ref, wl_ref, bl_ref, inv_ref, t_ref,
                          st_ref)
    return pl.pallas_call(
        body,
        grid=(n_nodes // _NB,),
        in_specs=[
            pl.BlockSpec((ca, _NB, _F), lambda i: (0, i, 0)),
            *stats_specs,
            pl.BlockSpec((_NB, h), lambda i: (i, 0)),
            pl.BlockSpec((_NB, 1), lambda i: (i, 0)),
            pl.BlockSpec((_NB, 1), lambda i: (i, 0)),
            pl.BlockSpec(wl.shape, lambda i: (0, 0)),
            pl.BlockSpec((1, h), lambda i: (0, 0)),
            pl.BlockSpec((1, _G), lambda i: (0, 0)),
        ],
        out_specs=[
            pl.BlockSpec((nch, _NB, _F), lambda i: (0, i, 0)),
            pl.BlockSpec((2, h), lambda i: (0, 0)),
        ],
        out_shape=[
            jax.ShapeDtypeStruct((nch, n_nodes, _F), jnp.float32),
            jax.ShapeDtypeStruct((2, h), jnp.float32),
        ],
    )(*args, xwr, cnt, batch2, wl, bl, inv)


def _norm_kernel(n_nodes, t_ref, st_ref, g_ref, b_ref, o_ref):
    nch = o_ref.shape[0]
    t = jnp.concatenate([t_ref[k] for k in range(nch)], axis=1)
    a, bb = _bn_affine(st_ref, g_ref, b_ref, n_nodes)
    y = t * a + bb
    for k in range(nch):
        o_ref[k] = y[:, k * _F:(k + 1) * _F]


def _norm(tch, st, gamma, beta, n_nodes, h):
    nch = h // _F
    return pl.pallas_call(
        functools.partial(_norm_kernel, float(n_nodes)),
        grid=(n_nodes // _NB,),
        in_specs=[
            pl.BlockSpec((nch, _NB, _F), lambda i: (0, i, 0)),
            pl.BlockSpec((2, h), lambda i: (0, 0)),
            pl.BlockSpec((1, h), lambda i: (0, 0)),
            pl.BlockSpec((1, h), lambda i: (0, 0)),
        ],
        out_specs=pl.BlockSpec((nch, _NB, _F), lambda i: (0, i, 0)),
        out_shape=jax.ShapeDtypeStruct((nch, n_nodes, _F), jnp.float32),
    )(tch, st, gamma, beta)


def _gate_kernel(x_ref, batch_ref, wg_ref, bg_ref, gate_ref, m_ref):
    i = pl.program_id(0)
    nc = x_ref.shape[0]
    x = jnp.concatenate([x_ref[k] for k in range(nc)], axis=1)
    g = jnp.dot(x, wg_ref[...], preferred_element_type=jnp.float32) + bg_ref[...]
    gate_ref[...] = g
    ohb = batch_ref[...] == lax.broadcasted_iota(jnp.int32, (_NB, _G), 1)
    wh = jnp.where(ohb, g, -jnp.inf)
    blkmax = jnp.max(wh, axis=0, keepdims=True)

    @pl.when(i == 0)
    def _():
        m_ref[...] = jnp.full_like(m_ref, -jnp.inf)

    m_ref[...] = jnp.maximum(m_ref[...], blkmax)


def _gate(xch, batch2, wg, bg, n_nodes, h):
    nch = h // _F
    return pl.pallas_call(
        _gate_kernel,
        grid=(n_nodes // _NB,),
        in_specs=[
            pl.BlockSpec((nch, _NB, _F), lambda i: (0, i, 0)),
            pl.BlockSpec((_NB, 1), lambda i: (i, 0)),
            pl.BlockSpec((h, 1), lambda i: (0, 0)),
            pl.BlockSpec((1, 1), lambda i: (0, 0)),
        ],
        out_specs=[
            pl.BlockSpec((_NB, 1), lambda i: (i, 0)),
            pl.BlockSpec((1, _G), lambda i: (0, 0)),
        ],
        out_shape=[
            jax.ShapeDtypeStruct((n_nodes, 1), jnp.float32),
            jax.ShapeDtypeStruct((1, _G), jnp.float32),
        ],
    )(xch, batch2, wg, bg)


def _pool_kernel(x_ref, batch_ref, gate_ref, m_ref, u_ref, s_ref):
    i = pl.program_id(0)
    nc = x_ref.shape[0]
    x = jnp.concatenate([x_ref[k] for k in range(nc)], axis=1)
    m = m_ref[...]
    m = jnp.where(jnp.isfinite(m), m, 0.0)
    oh = (batch_ref[...] == lax.broadcasted_iota(jnp.int32, (_NB, _G), 1)
          ).astype(jnp.float32)
    mg = jnp.sum(oh * m, axis=1, keepdims=True)
    e = jnp.exp(gate_ref[...] - mg)
    we = oh * e

    @pl.when(i == 0)
    def _():
        u_ref[...] = jnp.zeros_like(u_ref)
        s_ref[...] = jnp.zeros_like(s_ref)

    u_ref[...] += lax.dot_general(we, x, (((0,), (0,)), ((), ())),
                                  preferred_element_type=jnp.float32)
    ones = jnp.ones((_NB, 1), jnp.float32)
    s_ref[...] += lax.dot_general(we, ones, (((0,), (0,)), ((), ())),
                                  preferred_element_type=jnp.float32)


def _pool(xch, batch2, gate, m, n_nodes, h):
    nch = h // _F
    return pl.pallas_call(
        _pool_kernel,
        grid=(n_nodes // _NB,),
        in_specs=[
            pl.BlockSpec((nch, _NB, _F), lambda i: (0, i, 0)),
            pl.BlockSpec((_NB, 1), lambda i: (i, 0)),
            pl.BlockSpec((_NB, 1), lambda i: (i, 0)),
            pl.BlockSpec((1, _G), lambda i: (0, 0)),
        ],
        out_specs=[
            pl.BlockSpec((_G, h), lambda i: (0, 0)),
            pl.BlockSpec((_G, 1), lambda i: (0, 0)),
        ],
        out_shape=[
            jax.ShapeDtypeStruct((_G, h), jnp.float32),
            jax.ShapeDtypeStruct((_G, 1), jnp.float32),
        ],
    )(xch, batch2, gate, m)


def _mlp_kernel(u_ref, s_ref, w1_ref, b1_ref, w2_ref, b2_ref, o_ref):
    p = u_ref[...] / jnp.maximum(s_ref[...], 1e-16)
    hdn = jnp.maximum(
        jnp.dot(p, w1_ref[...], preferred_element_type=jnp.float32)
        + b1_ref[...], 0.0)
    o_ref[...] = (jnp.dot(hdn, w2_ref[...], preferred_element_type=jnp.float32)
                  + b2_ref[...])


def _mlp(u, s, w1, b1, w2, b2, h, dout):
    return pl.pallas_call(
        _mlp_kernel,
        out_shape=jax.ShapeDtypeStruct((_G, dout), jnp.float32),
    )(u, s, w1, b1, w2, b2)


def kernel(x, edge_index, batch, Wl0, bl0, Wr0, gamma0, beta0, Wl1, bl1, Wr1,
           gamma1, beta1, Wl2, bl2, Wr2, gamma2, beta2, Wg, bg, W1, b1, W2,
           b2):
    n, din = x.shape
    h = Wl0.shape[1]
    dout = W2.shape[1]
    src, dst = edge_index[0], edge_index[1]
    srcp, dstp, ep = _pad_edges(src, dst, n)
    batch2 = batch.reshape(n, 1)

    _, n_acc = _acc_rows(n)
    inv = _deg_inv(batch2, n)
    cnt = _make_count(n, ep)(dstp)[:n, :1]

    xch = x.reshape(n, din // _F, _F).transpose(1, 0, 2)

    layers = [(Wl0, bl0, Wr0, gamma0, beta0),
              (Wl1, bl1, Wr1, gamma1, beta1),
              (Wl2, bl2, Wr2, gamma2, beta2)]
    st, gam, bet = None, None, None
    for (wl, bl, wr, gamma, beta) in layers:
        c = xch.shape[0]
        aggf = _make_segsum(c, n, ep)(xch.reshape(c * n, _F), srcp, dstp)
        xwr = _xwr(xch, st, gam, bet, wr, n, h)
        agg = aggf.reshape(c, n_acc, _F)[:, :n, :]
        xch, st = _layer(agg, st, gam, bet, xwr, cnt, batch2, wl,
                         bl.reshape(1, h), inv, n, h)
        gam, bet = gamma.reshape(1, h), beta.reshape(1, h)

    xch = _norm(xch, st, gam, bet, n, h)
    gate, m = _gate(xch, batch2, Wg, bg.reshape(1, 1), n, h)
    u, s = _pool(xch, batch2, gate, m, n, h)
    return _mlp(u, s, W1, b1.reshape(1, h), W2, b2.reshape(1, dout), h, dout)
